# baseline (speedup 1.0000x reference)
import functools

import jax
import jax.numpy as jnp
from jax import lax
from jax.experimental import pallas as pl
from jax.experimental.pallas import tpu as pltpu
from jax.experimental.pallas import tpu_sc as plsc

D = 128
L = 16
NC = 2
NS = 16
NW = NC * NS
C = 16
NBUF = 6


@functools.lru_cache(maxsize=None)
def _make_sc_decoder(B: int, V: int):
    b_per_w = B // NW
    n_chunks = b_per_w // C
    mesh = plsc.VectorSubcoreMesh(core_axis_name="c", subcore_axis_name="s")

    @functools.partial(
        pl.kernel,
        mesh=mesh,
        out_type=jax.ShapeDtypeStruct((B,), jnp.float32),
        compiler_params=pltpu.CompilerParams(needs_layout_passes=False),
        scratch_types=[
            pltpu.VMEM((2 * b_per_w,), jnp.int32),
            [pltpu.VMEM((2 * C, D), jnp.float32) for _ in range(NBUF)],
            pltpu.VMEM((C,), jnp.float32),
            pltpu.VMEM_SHARED((V, D), jnp.float32),
            [pltpu.SemaphoreType.DMA for _ in range(NBUF)],
        ],
    )
    def body(z_hbm, idx_hbm, out_hbm, idx_v, bufs, out_v, z_sp, sems):
        wid = lax.axis_index("s") * NC + lax.axis_index("c")
        base = wid * b_per_w

        sid = lax.axis_index("s")
        n_stage = 10
        v_per_s = V // n_stage

        @pl.when(sid < n_stage)
        def _():
            soff = pl.multiple_of(sid * v_per_s, 8)
            pltpu.sync_copy(z_hbm.at[pl.ds(soff, v_per_s)],
                            z_sp.at[pl.ds(soff, v_per_s)])

        pltpu.sync_copy(idx_hbm.at[pl.ds(2 * base, 2 * b_per_w)], idx_v)
        plsc.subcore_barrier()

        def launch(ci, b):
            src = z_hbm
            pltpu.async_copy(
                src.at[idx_v.at[pl.ds(ci * 2 * C, 2 * C)]], bufs[b], sems[b])

        for b in range(NBUF):
            launch(b, b)

        iota = lax.iota(jnp.int32, L)

        def compute(ci, b):
            buf = bufs[b]
            pltpu.make_async_copy(z_hbm.at[idx_v.at[pl.ds(0, 2 * C)]],
                                  buf, sems[b]).wait()

            def t_body(t, acc):
                dcol = (iota + t) & (D - 1)
                a = plsc.load_gather(buf, [iota, dcol])
                b2 = plsc.load_gather(buf, [L + iota, dcol])
                return acc + a * b2

            acc = lax.fori_loop(0, D, t_body,
                                jnp.zeros((L,), jnp.float32), unroll=32)
            out_v[pl.ds(0, L)] = 1.0 / (1.0 + jnp.exp(-acc))
            pltpu.sync_copy(out_v, out_hbm.at[pl.ds(base + ci * C, C)])

        def outer_body(i, carry):
            for u in range(NBUF):
                ci = i * NBUF + u
                compute(ci, u)

                @pl.when(ci + NBUF < n_chunks)
                def _():
                    launch(ci + NBUF, u)
            return carry

        n_main = (n_chunks // NBUF) * NBUF
        lax.fori_loop(0, n_chunks // NBUF, outer_body, 0)
        for ci in range(n_main, n_chunks):
            compute(ci, ci % NBUF)

    return body


def kernel(z, edge_index):
    ei = edge_index.astype(jnp.int32)
    B = ei.shape[1]
    inter = jnp.stack(
        [ei[0].reshape(-1, C), ei[1].reshape(-1, C)], axis=1).reshape(-1)
    return _make_sc_decoder(B, z.shape[0])(z, inter)


# speedup vs baseline: 1.0461x; 1.0461x over previous
import functools

import jax
import jax.numpy as jnp
from jax import lax
from jax.experimental import pallas as pl
from jax.experimental.pallas import tpu as pltpu
from jax.experimental.pallas import tpu_sc as plsc

D = 128
L = 16
NC = 2
NS = 16
NW = NC * NS
C = 16
NBUF = 5


@functools.lru_cache(maxsize=None)
def _make_sc_decoder(B: int, V: int):
    b_per_w = B // NW
    n_chunks = b_per_w // C
    mesh = plsc.VectorSubcoreMesh(core_axis_name="c", subcore_axis_name="s")

    @functools.partial(
        pl.kernel,
        mesh=mesh,
        out_type=jax.ShapeDtypeStruct((B,), jnp.float32),
        compiler_params=pltpu.CompilerParams(needs_layout_passes=False),
        scratch_types=[
            pltpu.VMEM((2 * b_per_w,), jnp.int32),
            [pltpu.VMEM((2 * C, D), jnp.float32) for _ in range(NBUF)],
            pltpu.VMEM((b_per_w,), jnp.float32),
            pltpu.VMEM_SHARED((V, D), jnp.float32),
            [pltpu.SemaphoreType.DMA for _ in range(NBUF)],
        ],
    )
    def body(z_hbm, idx_hbm, out_hbm, idx_v, bufs, out_v, z_sp, sems):
        wid = lax.axis_index("s") * NC + lax.axis_index("c")
        base = wid * b_per_w

        sid = lax.axis_index("s")
        n_stage = 10
        v_per_s = V // n_stage

        @pl.when(sid < n_stage)
        def _():
            soff = pl.multiple_of(sid * v_per_s, 8)
            pltpu.sync_copy(z_hbm.at[pl.ds(soff, v_per_s)],
                            z_sp.at[pl.ds(soff, v_per_s)])

        pltpu.sync_copy(idx_hbm.at[pl.ds(2 * base, 2 * b_per_w)], idx_v)
        plsc.subcore_barrier()

        def launch(ci, b):
            src = z_hbm if b % 2 == 0 else z_sp
            pltpu.async_copy(
                src.at[idx_v.at[pl.ds(ci * 2 * C, 2 * C)]], bufs[b], sems[b])

        for b in range(NBUF):
            launch(b, b)

        iota = lax.iota(jnp.int32, L)

        def compute(ci, b):
            buf = bufs[b]
            pltpu.make_async_copy(z_hbm.at[idx_v.at[pl.ds(0, 2 * C)]],
                                  buf, sems[b]).wait()

            def t_body(t, acc):
                dcol = (iota + t) & (D - 1)
                a = plsc.load_gather(buf, [iota, dcol])
                b2 = plsc.load_gather(buf, [L + iota, dcol])
                return acc + a * b2

            acc = lax.fori_loop(0, D, t_body,
                                jnp.zeros((L,), jnp.float32), unroll=32)
            out_v[pl.ds(ci * C, L)] = 1.0 / (1.0 + jnp.exp(-acc))

        def outer_body(i, carry):
            for u in range(NBUF):
                ci = i * NBUF + u
                compute(ci, u)

                @pl.when(ci + NBUF < n_chunks)
                def _():
                    launch(ci + NBUF, u)
            return carry

        n_main = (n_chunks // NBUF) * NBUF
        lax.fori_loop(0, n_chunks // NBUF, outer_body, 0)
        for ci in range(n_main, n_chunks):
            compute(ci, ci % NBUF)
        pltpu.sync_copy(out_v, out_hbm.at[pl.ds(base, b_per_w)])

    return body


def kernel(z, edge_index):
    ei = edge_index.astype(jnp.int32)
    B = ei.shape[1]
    inter = jnp.stack(
        [ei[0].reshape(-1, C), ei[1].reshape(-1, C)], axis=1).reshape(-1)
    return _make_sc_decoder(B, z.shape[0])(z, inter)


# speedup vs baseline: 1.7526x; 1.6754x over previous
import functools

import jax
import jax.numpy as jnp
from jax import lax
from jax.experimental import pallas as pl
from jax.experimental.pallas import tpu as pltpu
from jax.experimental.pallas import tpu_sc as plsc

D = 128
L = 16
NC = 2
NS = 16
NW = NC * NS
C = 16
NBUF = 4


@functools.lru_cache(maxsize=None)
def _make_sc_decoder(B: int, V: int):
    b_per_w = B // NW
    n_chunks = b_per_w // C
    mesh = plsc.VectorSubcoreMesh(core_axis_name="c", subcore_axis_name="s")

    @functools.partial(
        pl.kernel,
        mesh=mesh,
        out_type=jax.ShapeDtypeStruct((B,), jnp.float32),
        compiler_params=pltpu.CompilerParams(needs_layout_passes=False),
        scratch_types=[
            pltpu.VMEM((b_per_w,), jnp.int32),
            pltpu.VMEM((b_per_w,), jnp.int32),
            [pltpu.VMEM((C, D), jnp.float32) for _ in range(NBUF)],
            [pltpu.VMEM((C, D), jnp.float32) for _ in range(NBUF)],
            pltpu.VMEM((b_per_w,), jnp.float32),
            pltpu.VMEM_SHARED((V, D), jnp.float32),
            [pltpu.SemaphoreType.DMA for _ in range(NBUF)],
            [pltpu.SemaphoreType.DMA for _ in range(NBUF)],
        ],
    )
    def body(z_hbm, row_hbm, col_hbm, out_hbm,
             ridx_v, cidx_v, rows_bufs, cols_bufs, out_v, z_sp, sems_r, sems_c):
        wid = lax.axis_index("s") * NC + lax.axis_index("c")
        base = wid * b_per_w

        sid = lax.axis_index("s")
        n_stage = 10
        v_per_s = V // n_stage

        @pl.when(sid < n_stage)
        def _():
            soff = pl.multiple_of(sid * v_per_s, 8)
            pltpu.sync_copy(z_hbm.at[pl.ds(soff, v_per_s)],
                            z_sp.at[pl.ds(soff, v_per_s)])

        pltpu.sync_copy(row_hbm.at[pl.ds(base, b_per_w)], ridx_v)
        pltpu.sync_copy(col_hbm.at[pl.ds(base, b_per_w)], cidx_v)
        plsc.subcore_barrier()

        def launch(ci, b):
            coff = ci * C
            src = z_hbm if b % 2 == 0 else z_sp
            pltpu.async_copy(
                src.at[ridx_v.at[pl.ds(coff, C)]], rows_bufs[b], sems_r[b])
            pltpu.async_copy(
                src.at[cidx_v.at[pl.ds(coff, C)]], cols_bufs[b], sems_c[b])

        for b in range(NBUF):
            launch(b, b)

        iota = lax.iota(jnp.int32, L)

        def compute(ci, b):
            rows_v, cols_v = rows_bufs[b], cols_bufs[b]
            pltpu.make_async_copy(z_hbm.at[ridx_v.at[pl.ds(0, C)]],
                                  rows_v, sems_r[b]).wait()
            pltpu.make_async_copy(z_hbm.at[cidx_v.at[pl.ds(0, C)]],
                                  cols_v, sems_c[b]).wait()

            def group_body(g, c2):
                eb = g * L
                lanes = eb + iota
                def t_body(t, acc):
                    dcol = (iota + t) & (D - 1)
                    a = plsc.load_gather(rows_v, [lanes, dcol])
                    b2 = plsc.load_gather(cols_v, [lanes, dcol])
                    return acc + a * b2

                acc = lax.fori_loop(0, D, t_body,
                                    jnp.zeros((L,), jnp.float32), unroll=32)
                out_v[pl.ds(ci * C + eb, L)] = 1.0 / (1.0 + jnp.exp(-acc))
                return c2

            lax.fori_loop(0, C // L, group_body, 0)

        def outer_body(i, carry):
            for b in range(NBUF):
                ci = i * NBUF + b
                compute(ci, b)

                @pl.when(ci + NBUF < n_chunks)
                def _():
                    launch(ci + NBUF, b)
            return carry

        n_main = (n_chunks // NBUF) * NBUF
        lax.fori_loop(0, n_chunks // NBUF, outer_body, 0)
        for ci in range(n_main, n_chunks):
            compute(ci, ci % NBUF)
        pltpu.sync_copy(out_v, out_hbm.at[pl.ds(base, b_per_w)])

    return body


def kernel(z, edge_index):
    ei = edge_index.astype(jnp.int32)
    return _make_sc_decoder(ei.shape[1], z.shape[0])(z, ei[0], ei[1])


# speedup vs baseline: 2.1341x; 1.2177x over previous
import functools

import jax
import jax.numpy as jnp
from jax import lax
from jax.experimental import pallas as pl
from jax.experimental.pallas import tpu as pltpu
from jax.experimental.pallas import tpu_sc as plsc

D = 128
L = 16
NC = 2
NS = 16
NW = NC * NS
C = 128
NBUF = 3


@functools.lru_cache(maxsize=None)
def _make_sc_decoder(B: int):
    b_per_w = B // NW
    n_chunks = b_per_w // C
    CT = b_per_w - n_chunks * C
    mesh = plsc.VectorSubcoreMesh(core_axis_name="c", subcore_axis_name="s")

    scratch = [
        pltpu.VMEM((b_per_w,), jnp.int32),
        pltpu.VMEM((b_per_w,), jnp.int32),
        [pltpu.VMEM((C, D), jnp.float32) for _ in range(NBUF)],
        [pltpu.VMEM((C, D), jnp.float32) for _ in range(NBUF)],
        pltpu.VMEM((C,), jnp.float32),
        [pltpu.SemaphoreType.DMA for _ in range(NBUF)],
        [pltpu.SemaphoreType.DMA for _ in range(NBUF)],
    ]
    if CT:
        scratch += [
            pltpu.VMEM((CT, D), jnp.float32),
            pltpu.VMEM((CT, D), jnp.float32),
            pltpu.SemaphoreType.DMA,
            pltpu.SemaphoreType.DMA,
        ]

    @functools.partial(
        pl.kernel,
        mesh=mesh,
        out_type=jax.ShapeDtypeStruct((B,), jnp.float32),
        compiler_params=pltpu.CompilerParams(needs_layout_passes=False),
        scratch_types=scratch,
    )
    def body(z_hbm, row_hbm, col_hbm, out_hbm,
             ridx_v, cidx_v, rows_bufs, cols_bufs, out_v, sems_r, sems_c,
             *tail_scratch):
        wid = lax.axis_index("s") * NC + lax.axis_index("c")
        base = wid * b_per_w
        pltpu.sync_copy(row_hbm.at[pl.ds(base, b_per_w)], ridx_v)
        pltpu.sync_copy(col_hbm.at[pl.ds(base, b_per_w)], cidx_v)

        def launch(ci, b):
            coff = ci * C
            pltpu.async_copy(
                z_hbm.at[ridx_v.at[pl.ds(coff, C)]], rows_bufs[b], sems_r[b])
            pltpu.async_copy(
                z_hbm.at[cidx_v.at[pl.ds(coff, C)]], cols_bufs[b], sems_c[b])

        if CT:
            rows_t, cols_t, sem_rt, sem_ct = tail_scratch
            toff = n_chunks * C
            pltpu.async_copy(
                z_hbm.at[ridx_v.at[pl.ds(toff, CT)]], rows_t, sem_rt)
            pltpu.async_copy(
                z_hbm.at[cidx_v.at[pl.ds(toff, CT)]], cols_t, sem_ct)

        for b in range(NBUF):
            launch(b, b)

        iota = lax.iota(jnp.int32, L)

        def dot_groups(ci, rows_v, cols_v, n_edges):
            def group_body(g, c2):
                eb = g * L
                lanes = eb + iota

                def t_body(t, acc):
                    dcol = (iota + t) & (D - 1)
                    a = plsc.load_gather(rows_v, [lanes, dcol])
                    b2 = plsc.load_gather(cols_v, [lanes, dcol])
                    return acc + a * b2

                acc = lax.fori_loop(0, D, t_body,
                                    jnp.zeros((L,), jnp.float32), unroll=32)
                out_v[pl.ds(eb, L)] = 1.0 / (1.0 + jnp.exp(-acc))
                return c2

            lax.fori_loop(0, n_edges // L, group_body, 0)
            pltpu.sync_copy(out_v.at[pl.ds(0, n_edges)],
                            out_hbm.at[pl.ds(base + ci * C, n_edges)])

        def compute(ci, b):
            rows_v, cols_v = rows_bufs[b], cols_bufs[b]
            pltpu.make_async_copy(z_hbm.at[ridx_v.at[pl.ds(0, C)]],
                                  rows_v, sems_r[b]).wait()
            pltpu.make_async_copy(z_hbm.at[cidx_v.at[pl.ds(0, C)]],
                                  cols_v, sems_c[b]).wait()
            dot_groups(ci, rows_v, cols_v, C)

        def outer_body(i, carry):
            for b in range(NBUF):
                ci = i * NBUF + b
                compute(ci, b)

                @pl.when(ci + NBUF < n_chunks)
                def _():
                    launch(ci + NBUF, b)
            return carry

        n_main = (n_chunks // NBUF) * NBUF
        lax.fori_loop(0, n_chunks // NBUF, outer_body, 0)
        for ci in range(n_main, n_chunks):
            compute(ci, ci % NBUF)

        if CT:
            pltpu.make_async_copy(z_hbm.at[ridx_v.at[pl.ds(0, CT)]],
                                  rows_t, sem_rt).wait()
            pltpu.make_async_copy(z_hbm.at[cidx_v.at[pl.ds(0, CT)]],
                                  cols_t, sem_ct).wait()
            dot_groups(n_chunks, rows_t, cols_t, CT)

    return body


def kernel(z, edge_index):
    ei = edge_index.astype(jnp.int32)
    return _make_sc_decoder(ei.shape[1])(z, ei[0], ei[1])


# speedup vs baseline: 2.1469x; 1.0060x over previous
import functools

import jax
import jax.numpy as jnp
from jax import lax
from jax.experimental import pallas as pl
from jax.experimental.pallas import tpu as pltpu
from jax.experimental.pallas import tpu_sc as plsc

D = 128
L = 16
NC = 2
NS = 16
NW = NC * NS
C = 80
NBUF = 4


@functools.lru_cache(maxsize=None)
def _make_sc_decoder(B: int):
    b_per_w = B // NW
    n_chunks = b_per_w // C
    mesh = plsc.VectorSubcoreMesh(core_axis_name="c", subcore_axis_name="s")

    @functools.partial(
        pl.kernel,
        mesh=mesh,
        out_type=jax.ShapeDtypeStruct((B,), jnp.float32),
        compiler_params=pltpu.CompilerParams(needs_layout_passes=False),
        scratch_types=[
            pltpu.VMEM((b_per_w,), jnp.int32),
            pltpu.VMEM((b_per_w,), jnp.int32),
            [pltpu.VMEM((C, D), jnp.float32) for _ in range(NBUF)],
            [pltpu.VMEM((C, D), jnp.float32) for _ in range(NBUF)],
            pltpu.VMEM((C,), jnp.float32),
            [pltpu.SemaphoreType.DMA for _ in range(NBUF)],
            [pltpu.SemaphoreType.DMA for _ in range(NBUF)],
        ],
    )
    def body(z_hbm, row_hbm, col_hbm, out_hbm,
             ridx_v, cidx_v, rows_bufs, cols_bufs, out_v, sems_r, sems_c):
        wid = lax.axis_index("s") * NC + lax.axis_index("c")
        base = wid * b_per_w
        pltpu.sync_copy(row_hbm.at[pl.ds(base, b_per_w)], ridx_v)
        pltpu.sync_copy(col_hbm.at[pl.ds(base, b_per_w)], cidx_v)

        def launch(ci, b):
            coff = ci * C
            pltpu.async_copy(
                z_hbm.at[ridx_v.at[pl.ds(coff, C)]], rows_bufs[b], sems_r[b])
            pltpu.async_copy(
                z_hbm.at[cidx_v.at[pl.ds(coff, C)]], cols_bufs[b], sems_c[b])

        for b in range(NBUF):
            launch(b, b)

        iota = lax.iota(jnp.int32, L)

        def compute(ci, b):
            rows_v, cols_v = rows_bufs[b], cols_bufs[b]
            pltpu.make_async_copy(z_hbm.at[ridx_v.at[pl.ds(0, C)]],
                                  rows_v, sems_r[b]).wait()
            pltpu.make_async_copy(z_hbm.at[cidx_v.at[pl.ds(0, C)]],
                                  cols_v, sems_c[b]).wait()

            def group_body(g, c2):
                eb = g * L
                lanes = eb + iota
                def t_body(t, acc):
                    dcol = (iota + t) & (D - 1)
                    a = plsc.load_gather(rows_v, [lanes, dcol])
                    b2 = plsc.load_gather(cols_v, [lanes, dcol])
                    return acc + a * b2

                acc = lax.fori_loop(0, D, t_body,
                                    jnp.zeros((L,), jnp.float32), unroll=16)
                out_v[pl.ds(eb, L)] = 1.0 / (1.0 + jnp.exp(-acc))
                return c2

            lax.fori_loop(0, C // L, group_body, 0)
            pltpu.sync_copy(out_v, out_hbm.at[pl.ds(base + ci * C, C)])

        def outer_body(i, carry):
            for b in range(NBUF):
                ci = i * NBUF + b
                compute(ci, b)

                @pl.when(ci + NBUF < n_chunks)
                def _():
                    launch(ci + NBUF, b)
            return carry

        n_main = (n_chunks // NBUF) * NBUF
        lax.fori_loop(0, n_chunks // NBUF, outer_body, 0)
        for ci in range(n_main, n_chunks):
            compute(ci, ci % NBUF)

    return body


def kernel(z, edge_index):
    ei = edge_index.astype(jnp.int32)
    return _make_sc_decoder(ei.shape[1])(z, ei[0], ei[1])
